# baseline (device time: 15071 ns/iter reference)
import jax
import jax.numpy as jnp
from jax import lax
from jax.experimental import pallas as pl
from jax.experimental.pallas import tpu as pltpu

N_DEV = 4


def kernel(ids, E):
    v_per, d = E.shape

    my_pos = lax.axis_index("i")
    offset = my_pos * v_per
    local = ids - offset
    in_range = (local >= 0) & (local < v_per)
    safe = jnp.where(in_range, local, 0)
    partial = jnp.take(E, safe, axis=0) * in_range[:, None].astype(E.dtype)
    return _direct_all_reduce(partial)


N_HALF = 2


def _direct_all_reduce(x):
    t, d = x.shape
    chunk = t // N_DEV
    dh = d // N_HALF

    def body(x_ref, out_ref, xb, rs_buf, ag_buf, red_bf,
             rs_send_sems, rs_recv_sems, ag_send_sems, ag_recv_sems):
        my = lax.axis_index("i")

        barrier_sem = pltpu.get_barrier_semaphore()
        for k in range(1, N_DEV):
            peer = lax.rem(my + k, N_DEV)
            pl.semaphore_signal(
                barrier_sem, inc=1,
                device_id=(peer,), device_id_type=pl.DeviceIdType.MESH,
            )
        pl.semaphore_wait(barrier_sem, N_DEV - 1)

        rs = [[None] * (N_DEV - 1) for _ in range(N_HALF)]
        for h in range(N_HALF):
            cs = pl.ds(h * dh, dh)
            xb[:, cs] = x_ref[:, cs].astype(jnp.bfloat16)
            for k in range(1, N_DEV):
                peer = lax.rem(my + k, N_DEV)
                r = pltpu.make_async_remote_copy(
                    src_ref=xb.at[pl.ds(peer * chunk, chunk), cs],
                    dst_ref=rs_buf.at[k - 1, :, cs],
                    send_sem=rs_send_sems.at[h, k - 1],
                    recv_sem=rs_recv_sems.at[h, k - 1],
                    device_id=(peer,),
                    device_id_type=pl.DeviceIdType.MESH,
                )
                r.start()
                rs[h][k - 1] = r
        ag = [[None] * (N_DEV - 1) for _ in range(N_HALF)]
        for h in range(N_HALF):
            cs = pl.ds(h * dh, dh)
            for r in rs[h]:
                r.wait_recv()
            acc = (x_ref[pl.ds(my * chunk, chunk), cs]
                   + rs_buf[0, :, cs].astype(jnp.float32)
                   + rs_buf[1, :, cs].astype(jnp.float32)
                   + rs_buf[2, :, cs].astype(jnp.float32))
            out_ref[pl.ds(my * chunk, chunk), cs] = acc
            red_bf[:, cs] = acc.astype(jnp.bfloat16)
            for k in range(1, N_DEV):
                peer = lax.rem(my + k, N_DEV)
                r = pltpu.make_async_remote_copy(
                    src_ref=red_bf.at[:, cs],
                    dst_ref=ag_buf.at[k - 1, :, cs],
                    send_sem=ag_send_sems.at[h, k - 1],
                    recv_sem=ag_recv_sems.at[h, k - 1],
                    device_id=(peer,),
                    device_id_type=pl.DeviceIdType.MESH,
                )
                r.start()
                ag[h][k - 1] = r
        for h in range(N_HALF):
            cs = pl.ds(h * dh, dh)
            for k in range(1, N_DEV):
                src = lax.rem(my - k + 2 * N_DEV, N_DEV)
                ag[h][k - 1].wait_recv()
                out_ref[pl.ds(src * chunk, chunk), cs] = (
                    ag_buf[k - 1, :, cs].astype(jnp.float32))
        for h in range(N_HALF):
            for r in rs[h]:
                r.wait_send()
            for r in ag[h]:
                r.wait_send()

    return pl.pallas_call(
        body,
        out_shape=jax.ShapeDtypeStruct((t, d), jnp.float32),
        in_specs=[pl.BlockSpec(memory_space=pltpu.VMEM)],
        out_specs=pl.BlockSpec(memory_space=pltpu.VMEM),
        scratch_shapes=[
            pltpu.VMEM((t, d), jnp.bfloat16),
            pltpu.VMEM((N_DEV - 1, chunk, d), jnp.bfloat16),
            pltpu.VMEM((N_DEV - 1, chunk, d), jnp.bfloat16),
            pltpu.VMEM((chunk, d), jnp.bfloat16),
            pltpu.SemaphoreType.DMA((N_HALF, N_DEV - 1)),
            pltpu.SemaphoreType.DMA((N_HALF, N_DEV - 1)),
            pltpu.SemaphoreType.DMA((N_HALF, N_DEV - 1)),
            pltpu.SemaphoreType.DMA((N_HALF, N_DEV - 1)),
        ],
        compiler_params=pltpu.CompilerParams(collective_id=0),
    )(x)


# device time: 14807 ns/iter; 1.0178x vs baseline; 1.0178x over previous
import jax
import jax.numpy as jnp
from jax import lax
from jax.experimental import pallas as pl
from jax.experimental.pallas import tpu as pltpu

N_DEV = 4


def kernel(ids, E):
    v_per, d = E.shape

    my_pos = lax.axis_index("i")
    offset = my_pos * v_per
    local = ids - offset
    in_range = (local >= 0) & (local < v_per)
    safe = jnp.where(in_range, local, 0)
    partial = jnp.take(E, safe, axis=0) * in_range[:, None].astype(E.dtype)
    return _direct_all_reduce(partial)


N_HALF = 4


def _direct_all_reduce(x):
    t, d = x.shape
    chunk = t // N_DEV
    dh = d // N_HALF

    def body(x_ref, out_ref, xb, rs_buf, ag_buf, red_bf,
             rs_send_sems, rs_recv_sems, ag_send_sems, ag_recv_sems):
        my = lax.axis_index("i")

        barrier_sem = pltpu.get_barrier_semaphore()
        for k in range(1, N_DEV):
            peer = lax.rem(my + k, N_DEV)
            pl.semaphore_signal(
                barrier_sem, inc=1,
                device_id=(peer,), device_id_type=pl.DeviceIdType.MESH,
            )
        pl.semaphore_wait(barrier_sem, N_DEV - 1)

        rs = [[None] * (N_DEV - 1) for _ in range(N_HALF)]
        for h in range(N_HALF):
            cs = pl.ds(h * dh, dh)
            xb[:, cs] = x_ref[:, cs].astype(jnp.bfloat16)
            for k in range(1, N_DEV):
                peer = lax.rem(my + k, N_DEV)
                r = pltpu.make_async_remote_copy(
                    src_ref=xb.at[pl.ds(peer * chunk, chunk), cs],
                    dst_ref=rs_buf.at[k - 1, :, cs],
                    send_sem=rs_send_sems.at[h, k - 1],
                    recv_sem=rs_recv_sems.at[h, k - 1],
                    device_id=(peer,),
                    device_id_type=pl.DeviceIdType.MESH,
                )
                r.start()
                rs[h][k - 1] = r
        ag = [[None] * (N_DEV - 1) for _ in range(N_HALF)]
        for h in range(N_HALF):
            cs = pl.ds(h * dh, dh)
            for r in rs[h]:
                r.wait_recv()
            acc = (x_ref[pl.ds(my * chunk, chunk), cs]
                   + rs_buf[0, :, cs].astype(jnp.float32)
                   + rs_buf[1, :, cs].astype(jnp.float32)
                   + rs_buf[2, :, cs].astype(jnp.float32))
            out_ref[pl.ds(my * chunk, chunk), cs] = acc
            red_bf[:, cs] = acc.astype(jnp.bfloat16)
            for k in range(1, N_DEV):
                peer = lax.rem(my + k, N_DEV)
                r = pltpu.make_async_remote_copy(
                    src_ref=red_bf.at[:, cs],
                    dst_ref=ag_buf.at[k - 1, :, cs],
                    send_sem=ag_send_sems.at[h, k - 1],
                    recv_sem=ag_recv_sems.at[h, k - 1],
                    device_id=(peer,),
                    device_id_type=pl.DeviceIdType.MESH,
                )
                r.start()
                ag[h][k - 1] = r
        for h in range(N_HALF):
            cs = pl.ds(h * dh, dh)
            for k in range(1, N_DEV):
                src = lax.rem(my - k + 2 * N_DEV, N_DEV)
                ag[h][k - 1].wait_recv()
                out_ref[pl.ds(src * chunk, chunk), cs] = (
                    ag_buf[k - 1, :, cs].astype(jnp.float32))
        for h in range(N_HALF):
            for r in rs[h]:
                r.wait_send()
            for r in ag[h]:
                r.wait_send()

    return pl.pallas_call(
        body,
        out_shape=jax.ShapeDtypeStruct((t, d), jnp.float32),
        in_specs=[pl.BlockSpec(memory_space=pltpu.VMEM)],
        out_specs=pl.BlockSpec(memory_space=pltpu.VMEM),
        scratch_shapes=[
            pltpu.VMEM((t, d), jnp.bfloat16),
            pltpu.VMEM((N_DEV - 1, chunk, d), jnp.bfloat16),
            pltpu.VMEM((N_DEV - 1, chunk, d), jnp.bfloat16),
            pltpu.VMEM((chunk, d), jnp.bfloat16),
            pltpu.SemaphoreType.DMA((N_HALF, N_DEV - 1)),
            pltpu.SemaphoreType.DMA((N_HALF, N_DEV - 1)),
            pltpu.SemaphoreType.DMA((N_HALF, N_DEV - 1)),
            pltpu.SemaphoreType.DMA((N_HALF, N_DEV - 1)),
        ],
        compiler_params=pltpu.CompilerParams(collective_id=0),
    )(x)
